# baseline (speedup 1.0000x reference)
import functools

import jax
import jax.numpy as jnp
from jax import lax
from jax.experimental import pallas as pl
from jax.experimental.pallas import tpu as pltpu
from jax.experimental.pallas import tpu_sc as plsc

N = 10000
E = 320000
D = 128

NC = 2
NS = 16
NW = NC * NS
CH = 128
NCHUNK = E // CH
CPT = NCHUNK // NW
EXTRA = NCHUNK - CPT * NW

ZCH = 5
ZSTRIDE = 632

_INV_BN = (1.0 + 1e-5) ** -0.5

_mesh = plsc.VectorSubcoreMesh(core_axis_name="c", subcore_axis_name="s")


@functools.partial(
    pl.kernel,
    out_type=jax.ShapeDtypeStruct((NC * N,), jnp.float32),
    mesh=_mesh,
    scratch_types=[
        pltpu.VMEM_SHARED((N,), jnp.float32),
        pltpu.VMEM((CPT * CH,), jnp.int32),
        pltpu.VMEM((CH,), jnp.int32),
        pltpu.VMEM((CH,), jnp.float32),
        pltpu.VMEM((CH,), jnp.float32),
        pltpu.VMEM((CH,), jnp.float32),
        pltpu.SemaphoreType.DMA,
    ],
)
def _deg_sc(dst_hbm, out_hbm, acc, didx, didx_x, ones_v, zero_v, wb_v, ss):
    c = lax.axis_index("c")
    s = lax.axis_index("s")
    wid = s * NC + c

    for k in range(CH // 16):
        ones_v[pl.ds(k * 16, 16)] = jnp.ones((16,), jnp.float32)
        zero_v[pl.ds(k * 16, 16)] = jnp.zeros((16,), jnp.float32)

    for k in range(ZCH):
        off = jnp.minimum(s * ZSTRIDE + k * CH, N - CH)
        pltpu.sync_copy(zero_v, acc.at[pl.ds(off, CH)])
    plsc.subcore_barrier()

    pltpu.sync_copy(dst_hbm.at[pl.ds(wid * CPT * CH, CPT * CH)], didx)

    K = 6
    def body(t, carry):
        for b in range(K):
            pltpu.async_copy(
                ones_v, acc.at[didx.at[pl.ds((t * K + b) * CH, CH)]], ss,
                add=True)
        for b in range(K):
            pltpu.make_async_copy(
                ones_v, acc.at[didx.at[pl.ds((t * K + b) * CH, CH)]],
                ss).wait()
        return carry

    lax.fori_loop(0, CPT // K, body, 0)
    for j in range(CPT - (CPT // K) * K):
        pltpu.sync_copy(
            ones_v, acc.at[didx.at[pl.ds(((CPT // K) * K + j) * CH, CH)]],
            add=True)

    @pl.when(wid < EXTRA)
    def _():
        xoff = (NCHUNK - EXTRA + wid) * CH
        pltpu.sync_copy(dst_hbm.at[pl.ds(xoff, CH)], didx_x)
        pltpu.sync_copy(ones_v, acc.at[didx_x], add=True)

    plsc.subcore_barrier()
    for k in range(ZCH):
        off = jnp.minimum(s * ZSTRIDE + k * CH, N - CH)
        pltpu.sync_copy(acc.at[pl.ds(off, CH)], wb_v)
        pltpu.sync_copy(wb_v, out_hbm.at[pl.ds(c * N + off, CH)])


_UNROLL = 6
assert CPT % _UNROLL == 0


@functools.partial(
    pl.kernel,
    out_type=jax.ShapeDtypeStruct((NC, N, D), jnp.float32),
    mesh=_mesh,
    scratch_types=(
        [pltpu.VMEM_SHARED((N, D), jnp.float32)]
        + [pltpu.VMEM((CH,), jnp.int32)] * 6
        + [pltpu.VMEM((CH,), jnp.int32)] * 6
        + [pltpu.VMEM((CH, D), jnp.float32)] * 3
        + [pltpu.SemaphoreType.DMA] * 12
    ),
)
def _mp_sc(h_hbm, src_hbm, dst_hbm, out_hbm, acc, *bufs):
    sidx = bufs[0:6]
    didx = bufs[6:12]
    rows = bufs[12:15]
    sd = bufs[15:21]
    sg = bufs[21:24]
    ss = bufs[24:27]
    c = lax.axis_index("c")
    s = lax.axis_index("s")
    wid = s * NC + c
    base = wid * CPT * CH

    def zbody(r, carry):
        for k in range(D // 16):
            rows[0][r, pl.ds(k * 16, 16)] = jnp.zeros((16,), jnp.float32)
        return carry

    lax.fori_loop(0, CH, zbody, 0)
    for k in range(ZCH):
        off = jnp.minimum(s * ZSTRIDE + k * CH, N - CH)
        pltpu.sync_copy(rows[0], acc.at[pl.ds(off, CH)])
    plsc.subcore_barrier()

    def pf_idx(j, i):
        pltpu.async_copy(src_hbm.at[pl.ds(base + j * CH, CH)], sidx[i],
                         sd[i])
        pltpu.async_copy(dst_hbm.at[pl.ds(base + j * CH, CH)], didx[i],
                         sd[i])

    def wait_idx(j, i):
        pltpu.make_async_copy(src_hbm.at[pl.ds(base + j * CH, CH)],
                              sidx[i], sd[i]).wait()
        pltpu.make_async_copy(dst_hbm.at[pl.ds(base + j * CH, CH)],
                              didx[i], sd[i]).wait()

    for i in range(4):
        pf_idx(i, i)
    wait_idx(0, 0)
    pltpu.async_copy(h_hbm.at[sidx[0]], rows[0], sg[0])

    def body(tt, carry):
        for u in range(_UNROLL):
            j = _UNROLL * tt + u
            r = u % 3
            pltpu.make_async_copy(h_hbm.at[sidx[u]], rows[r], sg[r]).wait()
            pltpu.async_copy(rows[r], acc.at[didx[u]], ss[r], add=True)

            r2 = (u + 1) % 3
            i2 = (u + 2) % 6
            if u >= 2:
                pltpu.make_async_copy(rows[r2], acc.at[didx[u - 2]],
                                      ss[r2]).wait()
            else:
                @pl.when(tt > 0)
                def _():
                    pltpu.make_async_copy(rows[r2], acc.at[didx[u + 4]],
                                          ss[r2]).wait()

            @pl.when(j + 4 < CPT)
            def _():
                pf_idx(j + 4, (u + 4) % 6)

            @pl.when(j + 1 < CPT)
            def _():
                wait_idx(j + 1, (u + 1) % 6)
                pltpu.async_copy(h_hbm.at[sidx[(u + 1) % 6]], rows[r2],
                                 sg[r2])
        return carry

    lax.fori_loop(0, CPT // _UNROLL, body, 0)

    pltpu.make_async_copy(rows[(CPT - 2) % 3], acc.at[didx[(CPT - 2) % 6]],
                          ss[(CPT - 2) % 3]).wait()
    pltpu.make_async_copy(rows[(CPT - 1) % 3], acc.at[didx[(CPT - 1) % 6]],
                          ss[(CPT - 1) % 3]).wait()

    @pl.when(wid < EXTRA)
    def _():
        xoff = (NCHUNK - EXTRA + wid) * CH
        pltpu.sync_copy(src_hbm.at[pl.ds(xoff, CH)], sidx[0])
        pltpu.sync_copy(dst_hbm.at[pl.ds(xoff, CH)], didx[0])
        pltpu.sync_copy(h_hbm.at[sidx[0]], rows[0])
        pltpu.sync_copy(rows[0], acc.at[didx[0]], add=True)

    plsc.subcore_barrier()
    for k in range(ZCH):
        off = jnp.minimum(s * ZSTRIDE + k * CH, N - CH)
        pltpu.sync_copy(acc.at[pl.ds(off, CH)],
                        out_hbm.at[c, pl.ds(off, CH)])


R = 1000


def _dinv_of(cnt_ref):
    seg = cnt_ref[...]
    return lax.rsqrt(1.0 + seg[:, 0] + seg[:, 1])[:, None]


def _tc1_body(cnt_ref, x_ref, w1_ref, h1s_ref):
    h = jnp.dot(x_ref[...], w1_ref[...], preferred_element_type=jnp.float32)
    h1s_ref[...] = h * _dinv_of(cnt_ref)


def _tc2_body(cnt_ref, acc_ref, h1s_ref, b1_ref, g_ref, be_ref, w2_ref,
              h2s_ref):
    dinv = _dinv_of(cnt_ref)
    s1 = acc_ref[0] + acc_ref[1] + h1s_ref[...]
    gcn1 = s1 * dinv + b1_ref[...]
    hh = jnp.maximum(gcn1 * (g_ref[...] * _INV_BN) + be_ref[...], 0.0)
    h2s_ref[...] = jnp.dot(hh, w2_ref[...],
                           preferred_element_type=jnp.float32) * dinv


def _tc3_body(cnt_ref, acc_ref, h2s_ref, b2_ref, x_ref, out_ref):
    dinv = _dinv_of(cnt_ref)
    s2 = acc_ref[0] + acc_ref[1] + h2s_ref[...]
    out_ref[...] = s2 * dinv + b2_ref[...] + x_ref[...]


_cnt_spec = pl.BlockSpec((R, 2), lambda i: (i, 0))
_row_spec = pl.BlockSpec((R, D), lambda i: (i, 0))
_acc_spec = pl.BlockSpec((2, R, D), lambda i: (0, i, 0))
_w_spec = pl.BlockSpec((D, D), lambda i: (0, 0))
_vec_spec = pl.BlockSpec((1, D), lambda i: (0, 0))

_tc1 = pl.pallas_call(
    _tc1_body, grid=(N // R,),
    in_specs=[_cnt_spec, _row_spec, _w_spec],
    out_specs=_row_spec,
    out_shape=jax.ShapeDtypeStruct((N, D), jnp.float32),
)

_tc2 = pl.pallas_call(
    _tc2_body, grid=(N // R,),
    in_specs=[_cnt_spec, _acc_spec, _row_spec, _vec_spec, _vec_spec,
              _vec_spec, _w_spec],
    out_specs=_row_spec,
    out_shape=jax.ShapeDtypeStruct((N, D), jnp.float32),
)

_tc3 = pl.pallas_call(
    _tc3_body, grid=(N // R,),
    in_specs=[_cnt_spec, _acc_spec, _row_spec, _vec_spec, _row_spec],
    out_specs=_row_spec,
    out_shape=jax.ShapeDtypeStruct((N, D), jnp.float32),
)


def kernel(x, edge_index, W1, b1, gamma, beta, W2, b2):
    src = edge_index[0]
    dst = edge_index[1]
    cnt = _deg_sc(dst).reshape(NC, N).T
    h1s = _tc1(cnt, x, W1)
    acc1 = _mp_sc(h1s, src, dst)
    h2s = _tc2(cnt, acc1, h1s, b1.reshape(1, D), gamma.reshape(1, D),
               beta.reshape(1, D), W2)
    acc2 = _mp_sc(h2s, src, dst)
    out = _tc3(cnt, acc2, h2s, b2.reshape(1, D), x)
    return out


# speedup vs baseline: 1.2280x; 1.2280x over previous
import functools

import jax
import jax.numpy as jnp
from jax import lax
from jax.experimental import pallas as pl
from jax.experimental.pallas import tpu as pltpu
from jax.experimental.pallas import tpu_sc as plsc

N = 10000
E = 320000
D = 128

NC = 2
NS = 16
NW = NC * NS
CH = 128
NCHUNK = E // CH
CPT = NCHUNK // NW
EXTRA = NCHUNK - CPT * NW

ZCH = 5
ZSTRIDE = 632

_INV_BN = (1.0 + 1e-5) ** -0.5

_mesh = plsc.VectorSubcoreMesh(core_axis_name="c", subcore_axis_name="s")


@functools.partial(
    pl.kernel,
    out_type=jax.ShapeDtypeStruct((NC * N,), jnp.float32),
    mesh=_mesh,
    scratch_types=[
        pltpu.VMEM_SHARED((N,), jnp.float32),
        pltpu.VMEM((CPT * CH,), jnp.int32),
        pltpu.VMEM((CH,), jnp.int32),
        pltpu.VMEM((CH,), jnp.float32),
        pltpu.VMEM((CH,), jnp.float32),
        pltpu.VMEM((CH,), jnp.float32),
        pltpu.SemaphoreType.DMA,
    ],
)
def _deg_sc(e_hbm, out_hbm, acc, didx, didx_x, ones_v, zero_v, wb_v, ss):
    c = lax.axis_index("c")
    s = lax.axis_index("s")
    wid = s * NC + c

    for k in range(CH // 16):
        ones_v[pl.ds(k * 16, 16)] = jnp.ones((16,), jnp.float32)
        zero_v[pl.ds(k * 16, 16)] = jnp.zeros((16,), jnp.float32)

    for k in range(ZCH):
        off = jnp.minimum(s * ZSTRIDE + k * CH, N - CH)
        pltpu.sync_copy(zero_v, acc.at[pl.ds(off, CH)])
    plsc.subcore_barrier()

    pltpu.sync_copy(e_hbm.at[pl.ds(E + wid * CPT * CH, CPT * CH)], didx)

    K = 6
    def body(t, carry):
        for b in range(K):
            pltpu.async_copy(
                ones_v, acc.at[didx.at[pl.ds((t * K + b) * CH, CH)]], ss,
                add=True)
        for b in range(K):
            pltpu.make_async_copy(
                ones_v, acc.at[didx.at[pl.ds((t * K + b) * CH, CH)]],
                ss).wait()
        return carry

    lax.fori_loop(0, CPT // K, body, 0)
    for j in range(CPT - (CPT // K) * K):
        pltpu.sync_copy(
            ones_v, acc.at[didx.at[pl.ds(((CPT // K) * K + j) * CH, CH)]],
            add=True)

    @pl.when(wid < EXTRA)
    def _():
        xoff = E + (NCHUNK - EXTRA + wid) * CH
        pltpu.sync_copy(e_hbm.at[pl.ds(xoff, CH)], didx_x)
        pltpu.sync_copy(ones_v, acc.at[didx_x], add=True)

    plsc.subcore_barrier()
    for k in range(ZCH):
        off = jnp.minimum(s * ZSTRIDE + k * CH, N - CH)
        pltpu.sync_copy(acc.at[pl.ds(off, CH)], wb_v)
        pltpu.sync_copy(wb_v, out_hbm.at[pl.ds(c * N + off, CH)])


@functools.partial(
    pl.kernel,
    out_type=jax.ShapeDtypeStruct((NC, N, D), jnp.float32),
    mesh=_mesh,
    scratch_types=[
        pltpu.VMEM_SHARED((N, D), jnp.float32),
        pltpu.VMEM((CPT * CH,), jnp.int32),
        pltpu.VMEM((CH,), jnp.int32),
        pltpu.VMEM((CH,), jnp.int32),
        pltpu.VMEM((CH,), jnp.int32),
        pltpu.VMEM((CH, D), jnp.float32),
        pltpu.VMEM((CH, D), jnp.float32),
        pltpu.SemaphoreType.DMA,
        pltpu.SemaphoreType.DMA,
        pltpu.SemaphoreType.DMA,
        pltpu.SemaphoreType.DMA,
        pltpu.SemaphoreType.DMA,
        pltpu.SemaphoreType.DMA,
    ],
)
def _mp_sc(h_hbm, e_hbm, out_hbm,
           acc, sidx, didx0, didx1, sidx_x, rows0, rows1,
           sg0, sg1, ss0, ss1, sd0, sd1):
    c = lax.axis_index("c")
    s = lax.axis_index("s")
    wid = s * NC + c

    def zbody(r, carry):
        for k in range(D // 16):
            rows0[r, pl.ds(k * 16, 16)] = jnp.zeros((16,), jnp.float32)
        return carry

    lax.fori_loop(0, CH, zbody, 0)
    for k in range(ZCH):
        off = jnp.minimum(s * ZSTRIDE + k * CH, N - CH)
        pltpu.sync_copy(rows0, acc.at[pl.ds(off, CH)])
    plsc.subcore_barrier()

    pltpu.sync_copy(e_hbm.at[pl.ds(wid * CPT * CH, CPT * CH)], sidx)

    base = E + wid * CPT * CH
    slots = ((rows0, didx0, sg0, ss0, sd0), (rows1, didx1, sg1, ss1, sd1))

    pltpu.async_copy(e_hbm.at[pl.ds(base, CH)], didx0, sd0)
    pltpu.async_copy(e_hbm.at[pl.ds(base + CH, CH)], didx1, sd1)
    pltpu.async_copy(h_hbm.at[sidx.at[pl.ds(0, CH)]], rows0, sg0)
    pltpu.async_copy(h_hbm.at[sidx.at[pl.ds(CH, CH)]], rows1, sg1)

    def body(t, carry):
        for b in range(2):
            j = 2 * t + b
            rows_b, didx_b, sg, ss, sd = slots[b]
            pltpu.make_async_copy(h_hbm.at[sidx.at[pl.ds(j * CH, CH)]],
                                  rows_b, sg).wait()
            pltpu.make_async_copy(e_hbm.at[pl.ds(base + j * CH, CH)],
                                  didx_b, sd).wait()
            pltpu.async_copy(rows_b, acc.at[didx_b], ss, add=True)
            pltpu.make_async_copy(rows_b, acc.at[didx_b], ss).wait()

            @pl.when(j + 2 < CPT)
            def _():
                pltpu.async_copy(e_hbm.at[pl.ds(base + (j + 2) * CH, CH)],
                                 didx_b, sd)
                pltpu.async_copy(h_hbm.at[sidx.at[pl.ds((j + 2) * CH, CH)]],
                                 rows_b, sg)
        return carry

    lax.fori_loop(0, CPT // 2, body, 0)

    @pl.when(wid < EXTRA)
    def _():
        xoff = (NCHUNK - EXTRA + wid) * CH
        pltpu.sync_copy(e_hbm.at[pl.ds(xoff, CH)], sidx_x)
        pltpu.sync_copy(e_hbm.at[pl.ds(E + xoff, CH)], didx0)
        pltpu.sync_copy(h_hbm.at[sidx_x], rows0)
        pltpu.sync_copy(rows0, acc.at[didx0], add=True)

    plsc.subcore_barrier()
    for k in range(ZCH):
        off = jnp.minimum(s * ZSTRIDE + k * CH, N - CH)
        pltpu.sync_copy(acc.at[pl.ds(off, CH)],
                        out_hbm.at[c, pl.ds(off, CH)])


R = 2000


def _dinv_of(cnt_ref):
    seg = cnt_ref[...]
    return lax.rsqrt(1.0 + seg[:, 0] + seg[:, 1])[:, None]


def _tc1_body(cnt_ref, x_ref, w1_ref, h1s_ref):
    h = jnp.dot(x_ref[...], w1_ref[...], preferred_element_type=jnp.float32)
    h1s_ref[...] = h * _dinv_of(cnt_ref)


def _tc2_body(cnt_ref, acc_ref, h1s_ref, b1_ref, g_ref, be_ref, w2_ref,
              h2s_ref):
    dinv = _dinv_of(cnt_ref)
    s1 = acc_ref[0] + acc_ref[1] + h1s_ref[...]
    gcn1 = s1 * dinv + b1_ref[...]
    hh = jnp.maximum(gcn1 * (g_ref[...] * _INV_BN) + be_ref[...], 0.0)
    h2s_ref[...] = jnp.dot(hh, w2_ref[...],
                           preferred_element_type=jnp.float32) * dinv


def _tc3_body(cnt_ref, acc_ref, h2s_ref, b2_ref, x_ref, out_ref):
    dinv = _dinv_of(cnt_ref)
    s2 = acc_ref[0] + acc_ref[1] + h2s_ref[...]
    out_ref[...] = s2 * dinv + b2_ref[...] + x_ref[...]


_cnt_spec = pl.BlockSpec((R, 2), lambda i: (i, 0))
_row_spec = pl.BlockSpec((R, D), lambda i: (i, 0))
_acc_spec = pl.BlockSpec((2, R, D), lambda i: (0, i, 0))
_w_spec = pl.BlockSpec((D, D), lambda i: (0, 0))
_vec_spec = pl.BlockSpec((1, D), lambda i: (0, 0))

_tc1 = pl.pallas_call(
    _tc1_body, grid=(N // R,),
    in_specs=[_cnt_spec, _row_spec, _w_spec],
    out_specs=_row_spec,
    out_shape=jax.ShapeDtypeStruct((N, D), jnp.float32),
)

_tc2 = pl.pallas_call(
    _tc2_body, grid=(N // R,),
    in_specs=[_cnt_spec, _acc_spec, _row_spec, _vec_spec, _vec_spec,
              _vec_spec, _w_spec],
    out_specs=_row_spec,
    out_shape=jax.ShapeDtypeStruct((N, D), jnp.float32),
)

_tc3 = pl.pallas_call(
    _tc3_body, grid=(N // R,),
    in_specs=[_cnt_spec, _acc_spec, _row_spec, _vec_spec, _row_spec],
    out_specs=_row_spec,
    out_shape=jax.ShapeDtypeStruct((N, D), jnp.float32),
)


def kernel(x, edge_index, W1, b1, gamma, beta, W2, b2):
    eflat = edge_index.reshape(2 * E)
    cnt = _deg_sc(eflat).reshape(NC, N).T
    h1s = _tc1(cnt, x, W1)
    acc1 = _mp_sc(h1s, eflat)
    h2s = _tc2(cnt, acc1, h1s, b1.reshape(1, D), gamma.reshape(1, D),
               beta.reshape(1, D), W2)
    acc2 = _mp_sc(h2s, eflat)
    out = _tc3(cnt, acc2, h2s, b2.reshape(1, D), x)
    return out
